# Initial kernel scaffold; baseline (speedup 1.0000x reference)
import jax
import jax.numpy as jnp
from jax.experimental import pallas as pl


def kernel(x, adj, Ws, bs):
    raise NotImplementedError("write your pallas kernel here")


# speedup vs baseline: 1.0100x; 1.0100x over previous
import functools

import jax
import jax.numpy as jnp
from jax.experimental import pallas as pl
from jax.experimental.pallas import tpu as pltpu


def _pick_block(n, candidates):
    for c in candidates:
        if n % c == 0:
            return c
    return n


def _layer_body(adj_ref, h_ref, w_ref, b_ref, o_ref):
    t = jnp.dot(adj_ref[...], h_ref[...], preferred_element_type=jnp.float32)
    o_ref[...] = (
        jnp.dot(t, w_ref[...], preferred_element_type=jnp.float32) + b_ref[...]
    )


@functools.partial(jax.jit, static_argnames=("br",))
def _layer(adj, h, w, b2d, br):
    n, d = h.shape
    return pl.pallas_call(
        _layer_body,
        grid=(n // br,),
        in_specs=[
            pl.BlockSpec((br, n), lambda r: (r, 0)),
            pl.BlockSpec((n, d), lambda r: (0, 0)),
            pl.BlockSpec((d, d), lambda r: (0, 0)),
            pl.BlockSpec((1, d), lambda r: (0, 0)),
        ],
        out_specs=pl.BlockSpec((br, d), lambda r: (r, 0)),
        out_shape=jax.ShapeDtypeStruct((n, d), jnp.float32),
        compiler_params=pltpu.CompilerParams(
            dimension_semantics=("parallel",),
        ),
    )(adj, h, w, b2d)


def kernel(x, adj, Ws, bs):
    n, d = x.shape
    br = _pick_block(n, (400, 200, 80, 40, 16, 8))
    h = x
    for i in range(Ws.shape[0]):
        h = _layer(adj, h, Ws[i], bs[i].reshape(1, d), br)
    return h


# speedup vs baseline: 1.3200x; 1.3070x over previous
import functools

import jax
import jax.numpy as jnp
from jax.experimental import pallas as pl
from jax.experimental.pallas import tpu as pltpu


def _pick_block(n, candidates):
    for c in candidates:
        if n % c == 0:
            return c
    return n


def _layer_body(adj_ref, h_ref, w_ref, b_ref, o_ref):
    t = jnp.dot(adj_ref[...], h_ref[...], preferred_element_type=jnp.float32)
    o_ref[...] = (
        jnp.dot(t, w_ref[...], preferred_element_type=jnp.float32) + b_ref[...]
    )


@functools.partial(jax.jit, static_argnames=("br",))
def _layer(adj, h, w, b2d, br):
    n, d = h.shape
    return pl.pallas_call(
        _layer_body,
        grid=(n // br,),
        in_specs=[
            pl.BlockSpec((br, n), lambda r: (r, 0)),
            pl.BlockSpec((n, d), lambda r: (0, 0)),
            pl.BlockSpec((d, d), lambda r: (0, 0)),
            pl.BlockSpec((1, d), lambda r: (0, 0)),
        ],
        out_specs=pl.BlockSpec((br, d), lambda r: (r, 0)),
        out_shape=jax.ShapeDtypeStruct((n, d), jnp.float32),
        compiler_params=pltpu.CompilerParams(
            dimension_semantics=("parallel",),
        ),
    )(adj, h, w, b2d)


def kernel(x, adj, Ws, bs):
    n, d = x.shape
    br = _pick_block(n, (400, 200, 80, 40, 16, 8))
    adj16 = adj.astype(jnp.bfloat16)
    h = x
    for i in range(Ws.shape[0]):
        h = _layer(adj16, h.astype(jnp.bfloat16), Ws[i], bs[i].reshape(1, d), br)
    return h


# speedup vs baseline: 1.4586x; 1.1050x over previous
import functools

import jax
import jax.numpy as jnp
from jax.experimental import pallas as pl
from jax.experimental.pallas import tpu as pltpu


def _pick_block(n, candidates):
    for c in candidates:
        if n % c == 0:
            return c
    return n


def _gcn_body(x_ref, adj_ref, ws_ref, bs_ref, o_ref, h_ref):
    l = pl.program_id(0)
    r = pl.program_id(1)
    nl = pl.num_programs(0)
    p = l % 2

    @pl.when(jnp.logical_and(l == 0, r == 0))
    def _():
        h_ref[0] = x_ref[...].astype(jnp.bfloat16)

    t = jnp.dot(adj_ref[...], h_ref[p], preferred_element_type=jnp.float32)
    out = (
        jnp.dot(t, ws_ref[l], preferred_element_type=jnp.float32)
        + bs_ref[l][None, :]
    )
    br = o_ref.shape[0]
    h_ref[1 - p, pl.ds(r * br, br), :] = out.astype(jnp.bfloat16)

    @pl.when(l == nl - 1)
    def _():
        o_ref[...] = out


@functools.partial(jax.jit, static_argnames=("br",))
def _gcn(x, adj16, Ws, bs, br):
    n, d = x.shape
    nl = Ws.shape[0]
    return pl.pallas_call(
        _gcn_body,
        grid=(nl, n // br),
        in_specs=[
            pl.BlockSpec((n, d), lambda l, r: (0, 0)),
            pl.BlockSpec((br, n), lambda l, r: (r, 0)),
            pl.BlockSpec((nl, d, d), lambda l, r: (0, 0, 0)),
            pl.BlockSpec((nl, d), lambda l, r: (0, 0)),
        ],
        out_specs=pl.BlockSpec(
            (br, d), lambda l, r: (jnp.where(l == nl - 1, r, 0), 0)
        ),
        out_shape=jax.ShapeDtypeStruct((n, d), jnp.float32),
        scratch_shapes=[pltpu.VMEM((2, n, d), jnp.bfloat16)],
        compiler_params=pltpu.CompilerParams(
            dimension_semantics=("arbitrary", "arbitrary"),
        ),
    )(x, adj16, Ws, bs)


def kernel(x, adj, Ws, bs):
    n, d = x.shape
    br = _pick_block(n, (400, 200, 80, 40, 16, 8))
    return _gcn(x, adj.astype(jnp.bfloat16), Ws, bs, br)


# speedup vs baseline: 1.5396x; 1.0555x over previous
import functools

import jax
import jax.numpy as jnp
from jax.experimental import pallas as pl
from jax.experimental.pallas import tpu as pltpu


def _pick_block(n, candidates):
    for c in candidates:
        if n % c == 0:
            return c
    return n


def _layer0_body(x16_ref, adj_ref, w_ref, b_ref, adj16_ref, h1_ref):
    a16 = adj_ref[...].astype(jnp.bfloat16)
    adj16_ref[...] = a16
    t = jnp.dot(a16, x16_ref[...], preferred_element_type=jnp.float32)
    out = (
        jnp.dot(t, w_ref[...], preferred_element_type=jnp.float32)
        + b_ref[...]
    )
    h1_ref[...] = out.astype(jnp.bfloat16)


def _rest_body(adj16_ref, h1_ref, ws_ref, bs_ref, o_ref, h_ref):
    l = pl.program_id(0)
    r = pl.program_id(1)
    nl = pl.num_programs(0)
    p = l % 2

    @pl.when(jnp.logical_and(l == 0, r == 0))
    def _():
        h_ref[0] = h1_ref[...]

    t = jnp.dot(adj16_ref[...], h_ref[p], preferred_element_type=jnp.float32)
    out = (
        jnp.dot(t, ws_ref[l], preferred_element_type=jnp.float32)
        + bs_ref[l][None, :]
    )
    br = o_ref.shape[0]
    h_ref[1 - p, pl.ds(r * br, br), :] = out.astype(jnp.bfloat16)

    @pl.when(l == nl - 1)
    def _():
        o_ref[...] = out


@functools.partial(jax.jit, static_argnames=("br0", "br"))
def _gcn(x, adj, Ws, bs, br0, br):
    n, d = x.shape
    nl = Ws.shape[0]
    x16 = x.astype(jnp.bfloat16)

    adj16, h1 = pl.pallas_call(
        _layer0_body,
        grid=(n // br0,),
        in_specs=[
            pl.BlockSpec((n, d), lambda r: (0, 0)),
            pl.BlockSpec((br0, n), lambda r: (r, 0)),
            pl.BlockSpec((d, d), lambda r: (0, 0)),
            pl.BlockSpec((1, d), lambda r: (0, 0)),
        ],
        out_specs=[
            pl.BlockSpec((br0, n), lambda r: (r, 0)),
            pl.BlockSpec((br0, d), lambda r: (r, 0)),
        ],
        out_shape=[
            jax.ShapeDtypeStruct((n, n), jnp.bfloat16),
            jax.ShapeDtypeStruct((n, d), jnp.bfloat16),
        ],
        compiler_params=pltpu.CompilerParams(
            dimension_semantics=("arbitrary",),
        ),
    )(x16, adj, Ws[0], bs[0].reshape(1, d))

    return pl.pallas_call(
        _rest_body,
        grid=(nl - 1, n // br),
        in_specs=[
            pl.BlockSpec((br, n), lambda l, r: (r, 0)),
            pl.BlockSpec((n, d), lambda l, r: (0, 0)),
            pl.BlockSpec((nl - 1, d, d), lambda l, r: (0, 0, 0)),
            pl.BlockSpec((nl - 1, d), lambda l, r: (0, 0)),
        ],
        out_specs=pl.BlockSpec(
            (br, d), lambda l, r: (jnp.where(l == nl - 2, r, 0), 0)
        ),
        out_shape=jax.ShapeDtypeStruct((n, d), jnp.float32),
        scratch_shapes=[pltpu.VMEM((2, n, d), jnp.bfloat16)],
        compiler_params=pltpu.CompilerParams(
            dimension_semantics=("arbitrary", "arbitrary"),
        ),
    )(adj16, h1, Ws[1:], bs[1:])


def kernel(x, adj, Ws, bs):
    n, _ = x.shape
    br0 = _pick_block(n, (200, 80, 40, 16, 8))
    br = _pick_block(n, (400, 200, 80, 40, 16, 8))
    return _gcn(x, adj, Ws, bs, br0, br)


# speedup vs baseline: 1.7601x; 1.1432x over previous
import functools

import jax
import jax.numpy as jnp
from jax.experimental import pallas as pl
from jax.experimental.pallas import tpu as pltpu


def _pick_block(n, candidates):
    for c in candidates:
        if n % c == 0:
            return c
    return n


def _layer0_body(x16_ref, adj_ref, w_ref, b_ref, adj16_ref, h1_ref):
    a16 = adj_ref[...].astype(jnp.bfloat16)
    adj16_ref[...] = a16
    t = jnp.dot(a16, x16_ref[...], preferred_element_type=jnp.float32)
    out = (
        jnp.dot(t, w_ref[...], preferred_element_type=jnp.float32)
        + b_ref[...]
    )
    h1_ref[...] = out.astype(jnp.bfloat16)


def _rest_body(adj16_ref, h1_ref, ws_ref, bs_ref, o_ref, h_ref):
    l = pl.program_id(0)
    r = pl.program_id(1)
    nl = pl.num_programs(0)
    p = l % 2

    @pl.when(jnp.logical_and(l == 0, r == 0))
    def _():
        h_ref[0] = h1_ref[...]

    t = jnp.dot(adj16_ref[...], h_ref[p], preferred_element_type=jnp.float32)
    out = (
        jnp.dot(t, ws_ref[l], preferred_element_type=jnp.float32)
        + bs_ref[l][None, :]
    )
    br = o_ref.shape[0]
    h_ref[1 - p, pl.ds(r * br, br), :] = out.astype(jnp.bfloat16)

    @pl.when(l == nl - 1)
    def _():
        o_ref[...] = out


@functools.partial(jax.jit, static_argnames=("br0", "br"))
def _gcn(x, adj, Ws, bs, br0, br):
    n, d = x.shape
    nl = Ws.shape[0]
    x16 = x.astype(jnp.bfloat16)

    adj16, h1 = pl.pallas_call(
        _layer0_body,
        grid=(n // br0,),
        in_specs=[
            pl.BlockSpec((n, d), lambda r: (0, 0)),
            pl.BlockSpec((br0, n), lambda r: (r, 0)),
            pl.BlockSpec((d, d), lambda r: (0, 0)),
            pl.BlockSpec((1, d), lambda r: (0, 0)),
        ],
        out_specs=[
            pl.BlockSpec((br0, n), lambda r: (r, 0)),
            pl.BlockSpec((br0, d), lambda r: (r, 0)),
        ],
        out_shape=[
            jax.ShapeDtypeStruct((n, n), jnp.bfloat16),
            jax.ShapeDtypeStruct((n, d), jnp.bfloat16),
        ],
        compiler_params=pltpu.CompilerParams(
            dimension_semantics=("arbitrary",),
        ),
    )(x16, adj, Ws[0], bs[0].reshape(1, d))

    return pl.pallas_call(
        _rest_body,
        grid=(nl - 1, n // br),
        in_specs=[
            pl.BlockSpec((br, n), lambda l, r: (r, 0)),
            pl.BlockSpec((n, d), lambda l, r: (0, 0)),
            pl.BlockSpec((nl - 1, d, d), lambda l, r: (0, 0, 0)),
            pl.BlockSpec((nl - 1, d), lambda l, r: (0, 0)),
        ],
        out_specs=pl.BlockSpec(
            (br, d), lambda l, r: (jnp.where(l == nl - 2, r, 0), 0)
        ),
        out_shape=jax.ShapeDtypeStruct((n, d), jnp.float32),
        scratch_shapes=[pltpu.VMEM((2, n, d), jnp.bfloat16)],
        compiler_params=pltpu.CompilerParams(
            dimension_semantics=("arbitrary", "arbitrary"),
        ),
    )(adj16, h1, Ws[1:], bs[1:])


def kernel(x, adj, Ws, bs):
    n, _ = x.shape
    br0 = _pick_block(n, (200, 80, 40, 16, 8))
    br = _pick_block(n, (1000, 400, 200, 80, 40, 16, 8))
    return _gcn(x, adj, Ws, bs, br0, br)
